# baseline (device time: 14601 ns/iter reference)
import jax
import jax.numpy as jnp
from jax import lax
from jax.experimental import pallas as pl
from jax.experimental.pallas import tpu as pltpu

CHUNKS = (128, 64, 32, 32)


def kernel(partial, resid, gamma):
    _, M, D = partial.shape
    MB = M // 2
    C = len(CHUNKS)
    OFFS = [sum(CHUNKS[:i]) for i in range(C)]

    def body(partial_ref, resid_ref, gamma_ref, out_ref,
             ysend, yrecv, ysend_sems, yrecv_sems, xsend_sems, xrecv_sems):
        my_x = lax.axis_index("x")
        my_y = lax.axis_index("y")
        ynbr = (my_x, 1 - my_y)
        xnbr = (1 - my_x, my_y)

        barrier_sem = pltpu.get_barrier_semaphore()
        for nbr in (ynbr, xnbr):
            pl.semaphore_signal(barrier_sem, inc=1, device_id=nbr,
                                device_id_type=pl.DeviceIdType.MESH)

        blk = my_x * MB
        ysend[...] = partial_ref[0, pl.ds(blk, MB), :].astype(jnp.bfloat16)

        pl.semaphore_wait(barrier_sem, 2)

        y_rdmas = []
        for c in range(C):
            r = pltpu.make_async_remote_copy(
                src_ref=ysend.at[pl.ds(OFFS[c], CHUNKS[c])],
                dst_ref=yrecv.at[pl.ds(OFFS[c], CHUNKS[c])],
                send_sem=ysend_sems.at[c],
                recv_sem=yrecv_sems.at[c],
                device_id=ynbr,
                device_id_type=pl.DeviceIdType.MESH,
            )
            r.start()
            y_rdmas.append(r)

        x_rdmas = []
        for c in range(C):
            y_rdmas[c].wait_recv()
            yv = (partial_ref[0, pl.ds(blk + OFFS[c], CHUNKS[c]), :]
                  + yrecv[pl.ds(OFFS[c], CHUNKS[c]), :].astype(jnp.float32)
                  + resid_ref[pl.ds(blk + OFFS[c], CHUNKS[c]), :])
            rms = jnp.sqrt(jnp.mean(yv * yv, axis=-1, keepdims=True) + 1e-6)
            out_ref[pl.ds(blk + OFFS[c], CHUNKS[c]), :] = (
                (yv / rms * gamma_ref[...]).astype(jnp.bfloat16))
            r = pltpu.make_async_remote_copy(
                src_ref=out_ref.at[pl.ds(blk + OFFS[c], CHUNKS[c])],
                dst_ref=out_ref.at[pl.ds(blk + OFFS[c], CHUNKS[c])],
                send_sem=xsend_sems.at[c],
                recv_sem=xrecv_sems.at[c],
                device_id=xnbr,
                device_id_type=pl.DeviceIdType.MESH,
            )
            r.start()
            x_rdmas.append(r)

        for c in range(C):
            x_rdmas[c].wait_recv()

        for c in range(C):
            y_rdmas[c].wait_send()
            x_rdmas[c].wait_send()

    return pl.pallas_call(
        body,
        out_shape=jax.ShapeDtypeStruct((M, D), jnp.bfloat16),
        in_specs=[pl.BlockSpec(memory_space=pltpu.VMEM)] * 3,
        out_specs=pl.BlockSpec(memory_space=pltpu.VMEM),
        scratch_shapes=[
            pltpu.VMEM((MB, D), jnp.bfloat16),
            pltpu.VMEM((MB, D), jnp.bfloat16),
            pltpu.SemaphoreType.DMA((C,)),
            pltpu.SemaphoreType.DMA((C,)),
            pltpu.SemaphoreType.DMA((C,)),
            pltpu.SemaphoreType.DMA((C,)),
        ],
        compiler_params=pltpu.CompilerParams(collective_id=0),
    )(partial, resid, gamma.reshape(1, D))


# device time: 12977 ns/iter; 1.1251x vs baseline; 1.1251x over previous
import jax
import jax.numpy as jnp
from jax import lax
from jax.experimental import pallas as pl
from jax.experimental.pallas import tpu as pltpu

S = 336
F = 176
OV = 160
F_CHUNKS = (16, 48, 48, 64)
OV_CHUNKS = (80, 80)


def kernel(partial, resid, gamma):
    _, M, D = partial.shape
    NF = len(F_CHUNKS)
    NO = len(OV_CHUNKS)
    F_OFFS = [sum(F_CHUNKS[:i]) for i in range(NF)]
    OV_OFFS = [F + sum(OV_CHUNKS[:i]) for i in range(NO)]

    def body(partial_ref, resid_ref, gamma_ref, out_ref,
             ysend, yrecv, xsend, xrecv,
             ysend_sems, yrecv_sems, xsend_sems, xrecv_sems):
        my_x = lax.axis_index("x")
        my_y = lax.axis_index("y")
        ynbr = (my_x, 1 - my_y)
        xnbr = (1 - my_x, my_y)

        barrier_sem = pltpu.get_barrier_semaphore()
        for nbr in (ynbr, xnbr):
            pl.semaphore_signal(barrier_sem, inc=1, device_id=nbr,
                                device_id_type=pl.DeviceIdType.MESH)

        fstart = my_x * S
        ostart = (1 - my_x) * S

        ysend[pl.ds(0, F), :] = (
            partial_ref[0, pl.ds(fstart, F), :].astype(jnp.bfloat16))
        ysend[pl.ds(F, OV), :] = (
            partial_ref[0, pl.ds(F, OV), :].astype(jnp.bfloat16))

        pl.semaphore_wait(barrier_sem, 2)

        y_rdmas = []
        for i, (off, sz) in enumerate(
                list(zip(F_OFFS, F_CHUNKS)) + list(zip(OV_OFFS, OV_CHUNKS))):
            r = pltpu.make_async_remote_copy(
                src_ref=ysend.at[pl.ds(off, sz)],
                dst_ref=yrecv.at[pl.ds(off, sz)],
                send_sem=ysend_sems.at[i],
                recv_sem=yrecv_sems.at[i],
                device_id=ynbr,
                device_id_type=pl.DeviceIdType.MESH,
            )
            r.start()
            y_rdmas.append(r)

        def reduce_norm(goff, boff, sz):
            yv = (partial_ref[0, pl.ds(goff, sz), :]
                  + yrecv[pl.ds(boff, sz), :].astype(jnp.float32)
                  + resid_ref[pl.ds(goff, sz), :])
            rms = jnp.sqrt(jnp.mean(yv * yv, axis=-1, keepdims=True) + 1e-6)
            o = (yv / rms * gamma_ref[...]).astype(jnp.bfloat16)
            out_ref[pl.ds(goff, sz), :] = o
            return o

        x_rdmas = []
        for c in range(NF):
            off, sz = F_OFFS[c], F_CHUNKS[c]
            y_rdmas[c].wait_recv()
            xsend[pl.ds(off, sz), :] = reduce_norm(fstart + off, off, sz)
            r = pltpu.make_async_remote_copy(
                src_ref=xsend.at[pl.ds(off, sz)],
                dst_ref=xrecv.at[pl.ds(off, sz)],
                send_sem=xsend_sems.at[c],
                recv_sem=xrecv_sems.at[c],
                device_id=xnbr,
                device_id_type=pl.DeviceIdType.MESH,
            )
            r.start()
            x_rdmas.append(r)

        for c in range(NO):
            boff, sz = OV_OFFS[c], OV_CHUNKS[c]
            y_rdmas[NF + c].wait_recv()
            reduce_norm(boff, boff, sz)

        for c in range(NF):
            off, sz = F_OFFS[c], F_CHUNKS[c]
            x_rdmas[c].wait_recv()
            out_ref[pl.ds(ostart + off, sz), :] = xrecv[pl.ds(off, sz), :]

        for r in y_rdmas + x_rdmas:
            r.wait_send()

    return pl.pallas_call(
        body,
        out_shape=jax.ShapeDtypeStruct((M, D), jnp.bfloat16),
        in_specs=[pl.BlockSpec(memory_space=pltpu.VMEM)] * 3,
        out_specs=pl.BlockSpec(memory_space=pltpu.VMEM),
        scratch_shapes=[
            pltpu.VMEM((S, D), jnp.bfloat16),
            pltpu.VMEM((S, D), jnp.bfloat16),
            pltpu.VMEM((F, D), jnp.bfloat16),
            pltpu.VMEM((F, D), jnp.bfloat16),
            pltpu.SemaphoreType.DMA((NF + NO,)),
            pltpu.SemaphoreType.DMA((NF + NO,)),
            pltpu.SemaphoreType.DMA((NF,)),
            pltpu.SemaphoreType.DMA((NF,)),
        ],
        compiler_params=pltpu.CompilerParams(collective_id=0),
    )(partial, resid, gamma.reshape(1, D))


# device time: 12952 ns/iter; 1.1273x vs baseline; 1.0019x over previous
import jax
import jax.numpy as jnp
from jax import lax
from jax.experimental import pallas as pl
from jax.experimental.pallas import tpu as pltpu

S = 336
F = 176
OV = 160
F_CHUNKS = (16, 32, 64, 64)
OV_CHUNKS = (80, 80)


def kernel(partial, resid, gamma):
    _, M, D = partial.shape
    NF = len(F_CHUNKS)
    NO = len(OV_CHUNKS)
    F_OFFS = [sum(F_CHUNKS[:i]) for i in range(NF)]
    OV_OFFS = [F + sum(OV_CHUNKS[:i]) for i in range(NO)]

    def body(partial_ref, resid_ref, gamma_ref, out_ref,
             ysend, yrecv, xsend, xrecv,
             ysend_sems, yrecv_sems, xsend_sems, xrecv_sems):
        my_x = lax.axis_index("x")
        my_y = lax.axis_index("y")
        ynbr = (my_x, 1 - my_y)
        xnbr = (1 - my_x, my_y)

        barrier_sem = pltpu.get_barrier_semaphore()
        for nbr in (ynbr, xnbr):
            pl.semaphore_signal(barrier_sem, inc=1, device_id=nbr,
                                device_id_type=pl.DeviceIdType.MESH)

        fstart = my_x * S
        ostart = (1 - my_x) * S

        ysend[pl.ds(0, F), :] = (
            partial_ref[0, pl.ds(fstart, F), :].astype(jnp.bfloat16))
        ysend[pl.ds(F, OV), :] = (
            partial_ref[0, pl.ds(F, OV), :].astype(jnp.bfloat16))

        pl.semaphore_wait(barrier_sem, 2)

        y_rdmas = []
        for i, (off, sz) in enumerate(
                list(zip(F_OFFS, F_CHUNKS)) + list(zip(OV_OFFS, OV_CHUNKS))):
            r = pltpu.make_async_remote_copy(
                src_ref=ysend.at[pl.ds(off, sz)],
                dst_ref=yrecv.at[pl.ds(off, sz)],
                send_sem=ysend_sems.at[i],
                recv_sem=yrecv_sems.at[i],
                device_id=ynbr,
                device_id_type=pl.DeviceIdType.MESH,
            )
            r.start()
            y_rdmas.append(r)

        def reduce_norm(goff, boff, sz):
            yv = (partial_ref[0, pl.ds(goff, sz), :]
                  + yrecv[pl.ds(boff, sz), :].astype(jnp.float32)
                  + resid_ref[pl.ds(goff, sz), :])
            rms = jnp.sqrt(jnp.mean(yv * yv, axis=-1, keepdims=True) + 1e-6)
            o = (yv / rms * gamma_ref[...]).astype(jnp.bfloat16)
            out_ref[pl.ds(goff, sz), :] = o
            return o

        x_rdmas = []
        for c in range(NF):
            off, sz = F_OFFS[c], F_CHUNKS[c]
            y_rdmas[c].wait_recv()
            xsend[pl.ds(off, sz), :] = reduce_norm(fstart + off, off, sz)
            r = pltpu.make_async_remote_copy(
                src_ref=xsend.at[pl.ds(off, sz)],
                dst_ref=xrecv.at[pl.ds(off, sz)],
                send_sem=xsend_sems.at[c],
                recv_sem=xrecv_sems.at[c],
                device_id=xnbr,
                device_id_type=pl.DeviceIdType.MESH,
            )
            r.start()
            x_rdmas.append(r)

        for c in range(NO):
            boff, sz = OV_OFFS[c], OV_CHUNKS[c]
            y_rdmas[NF + c].wait_recv()
            reduce_norm(boff, boff, sz)

        for c in range(NF):
            off, sz = F_OFFS[c], F_CHUNKS[c]
            x_rdmas[c].wait_recv()
            out_ref[pl.ds(ostart + off, sz), :] = xrecv[pl.ds(off, sz), :]

        for r in y_rdmas + x_rdmas:
            r.wait_send()

    return pl.pallas_call(
        body,
        out_shape=jax.ShapeDtypeStruct((M, D), jnp.bfloat16),
        in_specs=[pl.BlockSpec(memory_space=pltpu.VMEM)] * 3,
        out_specs=pl.BlockSpec(memory_space=pltpu.VMEM),
        scratch_shapes=[
            pltpu.VMEM((S, D), jnp.bfloat16),
            pltpu.VMEM((S, D), jnp.bfloat16),
            pltpu.VMEM((F, D), jnp.bfloat16),
            pltpu.VMEM((F, D), jnp.bfloat16),
            pltpu.SemaphoreType.DMA((NF + NO,)),
            pltpu.SemaphoreType.DMA((NF + NO,)),
            pltpu.SemaphoreType.DMA((NF,)),
            pltpu.SemaphoreType.DMA((NF,)),
        ],
        compiler_params=pltpu.CompilerParams(collective_id=0),
    )(partial, resid, gamma.reshape(1, D))
